# baseline (speedup 1.0000x reference)
import functools

import jax
import jax.numpy as jnp
from jax import lax
from jax.experimental import pallas as pl
from jax.experimental.pallas import tpu as pltpu
from jax.experimental.pallas import tpu_sc as plsc

NC = 2
NS = 16
NW = NC * NS


def _convert_body(in_ref, out_ref):
    out_ref[...] = in_ref[...].astype(jnp.bfloat16)


def _make_convert(V, D):
    ROWS = 25000
    assert V % ROWS == 0
    return pl.pallas_call(
        _convert_body,
        grid=(V // ROWS,),
        in_specs=[pl.BlockSpec((ROWS, D), lambda i: (i, 0))],
        out_specs=pl.BlockSpec((ROWS, D), lambda i: (i, 0)),
        out_shape=jax.ShapeDtypeStruct((V, D), jnp.bfloat16),
    )


def _make_sc_pool(B, H, D, CHUNK):
    CPB = H // CHUNK
    BPW = B // NW
    CW = BPW * CPB
    HALF = D // 2
    NBUF = 4
    assert CW % NBUF == 0 and CPB == 2 and D == 32

    mesh = plsc.VectorSubcoreMesh(
        core_axis_name="c", subcore_axis_name="s",
        num_cores=NC, num_subcores=NS)

    @functools.partial(
        pl.kernel,
        out_type=jax.ShapeDtypeStruct((B, D), jnp.float32),
        mesh=mesh,
        scratch_types=[
            pltpu.VMEM((CW, CHUNK), jnp.int32),
            pltpu.VMEM((NBUF, CHUNK, D), jnp.bfloat16),
            pltpu.VMEM((BPW, D), jnp.float32),
            pltpu.SemaphoreType.DMA((NBUF,)),
        ],
        compiler_params=pltpu.CompilerParams(
            use_tc_tiling_on_sc=False, needs_layout_passes=False),
    )
    def sc_pool(idx_hbm, table_hbm, out_hbm, idx_v, rows_v, acc_v, sems):
        w = lax.axis_index("s") * NC + lax.axis_index("c")
        cbase = w * CW

        pltpu.sync_copy(idx_hbm.at[pl.ds(cbase, CW)], idx_v)

        zero = jnp.zeros((16,), jnp.float32)
        himask = jnp.full((16,), -65536, jnp.int32)

        def start(c, buf):
            pltpu.async_copy(
                table_hbm.at[idx_v.at[c]], rows_v.at[buf], sems.at[buf])

        def wait(buf):
            pltpu.make_async_copy(
                table_hbm.at[idx_v.at[0]], rows_v.at[buf],
                sems.at[buf]).wait()

        for buf in range(NBUF - 1):
            start(buf, buf)

        def group(g, carry):
            accs = None
            for b in range(NBUF):
                c = g * NBUF + b
                nxt = c + NBUF - 1
                nbuf = (b + NBUF - 1) % NBUF

                @pl.when(nxt < CW)
                def _():
                    start(nxt, nbuf)

                wait(b)
                if b % CPB == 0:
                    accs = (zero, zero, zero, zero)
                a0, a1, a2, a3 = accs
                for j in range(CHUNK // 2):
                    x = plsc.bitcast(rows_v[b, 2 * j, :], jnp.int32)
                    y = plsc.bitcast(rows_v[b, 2 * j + 1, :], jnp.int32)
                    a0 = a0 + plsc.bitcast(x << 16, jnp.float32)
                    a1 = a1 + plsc.bitcast(x & himask, jnp.float32)
                    a2 = a2 + plsc.bitcast(y << 16, jnp.float32)
                    a3 = a3 + plsc.bitcast(y & himask, jnp.float32)
                accs = (a0, a1, a2, a3)
                if b % CPB == CPB - 1:
                    bag = (g * NBUF + b) // CPB
                    acc_v[bag, pl.ds(0, 16)] = a0 + a2
                    acc_v[bag, pl.ds(HALF, 16)] = a1 + a3
            return carry

        lax.fori_loop(0, CW // NBUF, group, 0)

        pltpu.sync_copy(acc_v, out_hbm.at[pl.ds(w * BPW, BPW)])

    return sc_pool


def _tc_head(pooled_ref, wt_ref, b_ref, out_ref, *, inv_h):
    hidden = pooled_ref[...] * inv_h
    logits = jnp.dot(hidden, wt_ref[...],
                     preferred_element_type=jnp.float32) + b_ref[...]
    m = jnp.max(logits, axis=1, keepdims=True)
    e = jnp.exp(logits - m)
    lse = jnp.log(jnp.sum(e, axis=1, keepdims=True)) + m
    out_ref[...] = logits - lse


def kernel(input_bags, emb_table, W, b):
    B, H = input_bags.shape
    V, D = emb_table.shape
    C = W.shape[0]
    CHUNK = 100
    assert H % CHUNK == 0 and B % NW == 0 and D == 32

    table_bf = _make_convert(V, D)(emb_table)
    idx = input_bags.reshape(B * (H // CHUNK), CHUNK)
    pooled = _make_sc_pool(B, H, D, CHUNK)(idx, table_bf)

    perm = jnp.arange(D).reshape(D // 2, 2).T.reshape(D)
    wt = W.T[perm]

    head = pl.pallas_call(
        functools.partial(_tc_head, inv_h=1.0 / H),
        out_shape=jax.ShapeDtypeStruct((B, C), jnp.float32),
    )
    return head(pooled, wt, b.reshape(1, C))


# speedup vs baseline: 1.6903x; 1.6903x over previous
import functools

import jax
import jax.numpy as jnp
from jax import lax
from jax.experimental import pallas as pl
from jax.experimental.pallas import tpu as pltpu
from jax.experimental.pallas import tpu_sc as plsc

NC = 2
NS = 16
NW = NC * NS


def _make_sc_pool(B, H, D, CHUNK):
    CPB = H // CHUNK
    BPW = B // NW
    CW = BPW * CPB
    HALF = D // 2
    NBUF = 4
    assert CW % NBUF == 0 and CPB == 2

    mesh = plsc.VectorSubcoreMesh(
        core_axis_name="c", subcore_axis_name="s",
        num_cores=NC, num_subcores=NS)

    @functools.partial(
        pl.kernel,
        out_type=jax.ShapeDtypeStruct((B, D), jnp.float32),
        mesh=mesh,
        scratch_types=[
            pltpu.VMEM((CW, CHUNK), jnp.int32),
            pltpu.VMEM((NBUF, CHUNK, D), jnp.float32),
            pltpu.VMEM((BPW, D), jnp.float32),
            pltpu.SemaphoreType.DMA((NBUF,)),
        ],
        compiler_params=pltpu.CompilerParams(use_tc_tiling_on_sc=False),
    )
    def sc_pool(idx_hbm, table_hbm, out_hbm, idx_v, rows_v, acc_v, sems):
        w = lax.axis_index("s") * NC + lax.axis_index("c")
        cbase = w * CW

        pltpu.sync_copy(idx_hbm.at[pl.ds(cbase, CW)], idx_v)

        zero = jnp.zeros((16,), jnp.float32)

        def start(c, buf):
            pltpu.async_copy(
                table_hbm.at[idx_v.at[c]], rows_v.at[buf], sems.at[buf])

        def wait(buf):
            pltpu.make_async_copy(
                table_hbm.at[idx_v.at[0]], rows_v.at[buf],
                sems.at[buf]).wait()

        for buf in range(NBUF - 1):
            start(buf, buf)

        def group(g, carry):
            accs = None
            for b in range(NBUF):
                c = g * NBUF + b
                nxt = c + NBUF - 1
                nbuf = (b + NBUF - 1) % NBUF

                @pl.when(nxt < CW)
                def _():
                    start(nxt, nbuf)

                wait(b)
                if b % CPB == 0:
                    accs = (zero, zero, zero, zero)
                a0, a1, a2, a3 = accs
                for j in range(CHUNK // 2):
                    a0 = a0 + rows_v[b, 2 * j, pl.ds(0, 16)]
                    a1 = a1 + rows_v[b, 2 * j, pl.ds(HALF, 16)]
                    a2 = a2 + rows_v[b, 2 * j + 1, pl.ds(0, 16)]
                    a3 = a3 + rows_v[b, 2 * j + 1, pl.ds(HALF, 16)]
                accs = (a0, a1, a2, a3)
                if b % CPB == CPB - 1:
                    bag = (g * NBUF + b) // CPB
                    acc_v[bag, pl.ds(0, 16)] = a0 + a2
                    acc_v[bag, pl.ds(HALF, 16)] = a1 + a3
            return carry

        lax.fori_loop(0, CW // NBUF, group, 0)

        pltpu.sync_copy(acc_v, out_hbm.at[pl.ds(w * BPW, BPW)])

    return sc_pool


def _tc_head(pooled_ref, wt_ref, b_ref, out_ref, *, inv_h):
    hidden = pooled_ref[...] * inv_h
    logits = jnp.dot(hidden, wt_ref[...],
                     preferred_element_type=jnp.float32) + b_ref[...]
    m = jnp.max(logits, axis=1, keepdims=True)
    e = jnp.exp(logits - m)
    lse = jnp.log(jnp.sum(e, axis=1, keepdims=True)) + m
    out_ref[...] = logits - lse


def kernel(input_bags, emb_table, W, b):
    B, H = input_bags.shape
    V, D = emb_table.shape
    C = W.shape[0]
    CHUNK = 100
    assert H % CHUNK == 0 and B % NW == 0 and D == 32

    idx = input_bags.reshape(B * (H // CHUNK), CHUNK)
    pooled = _make_sc_pool(B, H, D, CHUNK)(idx, emb_table)

    head = pl.pallas_call(
        functools.partial(_tc_head, inv_h=1.0 / H),
        out_shape=jax.ShapeDtypeStruct((B, C), jnp.float32),
    )
    return head(pooled, W.T, b.reshape(1, C))
